# baseline (speedup 1.0000x reference)
import jax
import jax.numpy as jnp
from jax.experimental import pallas as pl

_IDX0 = 7
_STRIDE = 156
_NPAIRS = 64
_NUM_NODES = 10000
_W = 2 * _NUM_NODES

_BLOCK_ROWS = 64


def _body(x_ref, o_ref):
    o_ref[...] = jnp.zeros_like(o_ref)
    x = x_ref[...]
    for k in range(_NPAIRS):
        col = 2 * (_IDX0 + _STRIDE * k)
        o_ref[:, col:col + 2] = x[:, 2 * k:2 * k + 2]


def kernel(f_lat):
    rows = f_lat.shape[0]
    out = pl.pallas_call(
        _body,
        grid=(rows // _BLOCK_ROWS,),
        in_specs=[pl.BlockSpec((_BLOCK_ROWS, 128), lambda i: (i, 0))],
        out_specs=pl.BlockSpec((_BLOCK_ROWS, _W), lambda i: (i, 0)),
        out_shape=jax.ShapeDtypeStruct((rows, _W), f_lat.dtype),
    )(f_lat)
    return out.reshape(rows, _NUM_NODES, 2)
